# Initial kernel scaffold; baseline (speedup 1.0000x reference)
import jax
import jax.numpy as jnp
from jax.experimental import pallas as pl


def kernel(mol_a_node_features, mol_a_edge_features, mol_a_edges, mol_a_batch_indices, mol_b_node_features, mol_b_edge_features, mol_b_edges, mol_b_batch_indices, proposed_bonds, W_edge, b_edge, W_node, b_node, mlp_W0, mlp_b0, mlp_W1, mlp_b1, mlp_W2, mlp_b2, mlp_W3, mlp_b3, mlp_W4, mlp_b4, mlp_W5, mlp_b5):
    raise NotImplementedError("write your pallas kernel here")


# speedup vs baseline: 1.0359x; 1.0359x over previous
import jax
import jax.numpy as jnp
from jax.experimental import pallas as pl
from jax.experimental.pallas import tpu as pltpu

NUM_STEPS = 100


def _mlp_body(x1, x2, x3, W0, b0, W1, b1, W2, b2, W3, b3, W4, b4, W5, b5, out):
    h = jnp.concatenate([x1[...], x2[...], x3[...]], axis=1)
    for W, b in ((W0, b0), (W1, b1), (W2, b2), (W3, b3), (W4, b4)):
        h = jax.nn.relu(jnp.dot(h, W[...], preferred_element_type=jnp.float32) + b[...])
    logits = jnp.dot(h, W5[...], preferred_element_type=jnp.float32) + b5[...]
    best = logits[:, 0:1]
    idx = jnp.zeros_like(best, dtype=jnp.int32)
    for k in range(1, 4):
        cur = logits[:, k:k + 1]
        m = cur > best
        idx = jnp.where(m, k, idx)
        best = jnp.where(m, cur, best)
    out[...] = idx


def kernel(mol_a_node_features, mol_a_edge_features, mol_a_edges, mol_a_batch_indices,
           mol_b_node_features, mol_b_edge_features, mol_b_edges, mol_b_batch_indices,
           proposed_bonds, W_edge, b_edge, W_node, b_node,
           mlp_W0, mlp_b0, mlp_W1, mlp_b1, mlp_W2, mlp_b2,
           mlp_W3, mlp_b3, mlp_W4, mlp_b4, mlp_W5, mlp_b5):
    Na = mol_a_node_features.shape[0]
    Ea = mol_a_edges.shape[0]
    Eb = mol_b_edges.shape[0]
    nc = proposed_bonds.shape[1]

    pb0 = proposed_bonds[0].astype(jnp.int32)
    pb1 = (proposed_bonds[1] + Na).astype(jnp.int32)
    src = jnp.concatenate([mol_a_edges[:, 0].astype(jnp.int32),
                           mol_b_edges[:, 0].astype(jnp.int32) + Na, pb0, pb1])
    dst = jnp.concatenate([mol_a_edges[:, 1].astype(jnp.int32),
                           mol_b_edges[:, 1].astype(jnp.int32) + Na, pb1, pb0])
    ef = jnp.concatenate([mol_a_edge_features, mol_b_edge_features,
                          jnp.full((2 * nc, 1), 999.0, dtype=jnp.float32)], axis=0)
    nf = jnp.concatenate([mol_a_node_features, mol_b_node_features], axis=0)
    N = nf.shape[0]

    W1 = W_edge[:32]
    W2 = W_edge[32:64]
    Wn2 = W_node[5:]
    const_edge = ef * W_edge[64][None, :] + b_edge[None, :]
    const_node = nf @ W_node[:5] + b_node[None, :]

    nh0 = jnp.zeros((N, 32), dtype=jnp.float32)
    eh0 = jnp.zeros((src.shape[0], 64), dtype=jnp.float32)

    def step(carry, _):
        nh, _eh = carry
        P1 = nh @ W1
        P2 = nh @ W2
        eh = jax.nn.relu(P1[src] + P2[dst] + const_edge)
        agg = jax.ops.segment_sum(eh, dst, num_segments=N)
        nh = jax.nn.relu(agg @ Wn2 + const_node)
        return (nh, eh), None

    (nh, eh), _ = jax.lax.scan(step, (nh0, eh0), None, length=NUM_STEPS)

    off = Ea + Eb
    pbh = eh[off:off + nc] + eh[off + nc:off + 2 * nc]
    nhA = nh[pb0]
    nhB = nh[pb1]

    labels = pl.pallas_call(
        _mlp_body,
        out_shape=jax.ShapeDtypeStruct((nc, 1), jnp.int32),
    )(nhA, nhB, pbh, mlp_W0, mlp_b0, mlp_W1, mlp_b1, mlp_W2, mlp_b2,
      mlp_W3, mlp_b3, mlp_W4, mlp_b4, mlp_W5, mlp_b5)

    return jnp.concatenate([proposed_bonds.T.astype(labels.dtype), labels], axis=1)


# speedup vs baseline: 2.7917x; 2.6949x over previous
import functools

import jax
import jax.numpy as jnp
from jax import lax
from jax.experimental import pallas as pl
from jax.experimental.pallas import tpu as pltpu
from jax.experimental.pallas import tpu_sc as plsc

NUM_STEPS = 100
NCORES = 2
NSUB = 16
NW = NCORES * NSUB
EBLK = 128


def _make_edge_kernel(n_nodes, ep, rows_per_tile):
    chunk = rows_per_tile // 2
    mesh = plsc.VectorSubcoreMesh(core_axis_name="c", subcore_axis_name="s")

    @functools.partial(
        pl.kernel,
        mesh=mesh,
        out_type=jax.ShapeDtypeStruct((n_nodes, 64), jnp.float32),
        scratch_types=[
            pltpu.VMEM((80,), jnp.int32),
            pltpu.VMEM((EBLK,), jnp.int32),
            pltpu.VMEM((EBLK + 16,), jnp.int32),
            pltpu.VMEM((EBLK + 16,), jnp.float32),
            pltpu.VMEM((EBLK, 128), jnp.float32),
            pltpu.VMEM((chunk, 64), jnp.float32),
            pltpu.VMEM((chunk, 64), jnp.float32),
            pltpu.VMEM((64,), jnp.float32),
            pltpu.SemaphoreType.DMA,
        ],
    )
    def edge_kernel(p1_hbm, p2b_hbm, src_hbm, dst_hbm, ef_hbm, bounds_hbm,
                    zero_hbm, w64_hbm, agg_hbm,
                    bounds_v, srcv, dstv, efv, rows1, p2l, aggl, w64v, sem):
        wid = lax.axis_index("s") * NCORES + lax.axis_index("c")
        pltpu.sync_copy(bounds_hbm, bounds_v)
        pltpu.sync_copy(w64_hbm, w64v)
        w64r = [w64v[pl.ds(16 * k, 16)] for k in range(4)]

        for c in range(2):
            g = wid * 2 + c
            base = g * chunk
            pltpu.sync_copy(zero_hbm, aggl)
            pltpu.sync_copy(p2b_hbm.at[pl.ds(base, chunk)], p2l)
            bv = bounds_v[pl.ds(g, 16)]
            elo = bv[0]
            ehi = bv[1]
            abase = (elo // EBLK) * EBLK
            nblk = (ehi - abase + (EBLK - 1)) // EBLK

            def blk_body(kb, _):
                off = abase + kb * EBLK
                pltpu.sync_copy(src_hbm.at[pl.ds(off, EBLK)], srcv)
                pltpu.sync_copy(dst_hbm.at[pl.ds(off, EBLK)],
                                dstv.at[pl.ds(0, EBLK)])
                pltpu.sync_copy(ef_hbm.at[pl.ds(off, EBLK)],
                                efv.at[pl.ds(0, EBLK)])
                pltpu.async_copy(p1_hbm.at[srcv], rows1, sem).wait()

                def e_body(j, _):
                    e = off + j
                    valid = jnp.logical_and(e >= elo, e < ehi)

                    @pl.when(valid)
                    def _():
                        d = dstv[pl.ds(j, 16)][0] - base
                        efs = efv[pl.ds(j, 16)][0]
                        for k in range(4):
                            r1 = rows1[j, pl.ds(16 * k, 16)]
                            r2 = p2l[d, pl.ds(16 * k, 16)]
                            r = jnp.maximum(r1 + r2 + efs * w64r[k], 0.0)
                            plsc.addupdate(aggl.at[d, pl.ds(16 * k, 16)], r)

                    return 0

                lax.fori_loop(0, EBLK, e_body, 0)
                return 0

            lax.fori_loop(0, nblk, blk_body, 0)
            pltpu.sync_copy(aggl, agg_hbm.at[pl.ds(base, chunk)])

    return edge_kernel


def _make_assemble_kernel(nc):
    rows = nc // NW
    mesh = plsc.VectorSubcoreMesh(core_axis_name="c", subcore_axis_name="s")

    @functools.partial(
        pl.kernel,
        mesh=mesh,
        out_type=jax.ShapeDtypeStruct((nc, 128), jnp.float32),
        scratch_types=[
            pltpu.VMEM((rows,), jnp.int32),
            pltpu.VMEM((rows,), jnp.int32),
            pltpu.VMEM((rows, 128), jnp.float32),
            pltpu.VMEM((rows, 128), jnp.float32),
            pltpu.VMEM((rows, 128), jnp.float32),
            pltpu.VMEM((rows, 128), jnp.float32),
            pltpu.VMEM((64,), jnp.float32),
            pltpu.VMEM((rows, 128), jnp.float32),
            pltpu.SemaphoreType.DMA,
        ],
    )
    def assemble_kernel(p1_hbm, q_hbm, pb0_hbm, pb1_hbm, w64_hbm,
                        out_hbm, ia, ib, g1, g3, qa, qb, w64v, obuf,
                        sem):
        wid = lax.axis_index("s") * NCORES + lax.axis_index("c")
        base = wid * rows
        pltpu.sync_copy(pb0_hbm.at[pl.ds(base, rows)], ia)
        pltpu.sync_copy(pb1_hbm.at[pl.ds(base, rows)], ib)
        pltpu.sync_copy(w64_hbm, w64v)
        pltpu.async_copy(p1_hbm.at[ia], g1, sem).wait()
        pltpu.async_copy(p1_hbm.at[ib], g3, sem).wait()
        pltpu.async_copy(q_hbm.at[ia], qa, sem).wait()
        pltpu.async_copy(q_hbm.at[ib], qb, sem).wait()
        c999 = [w64v[pl.ds(16 * k, 16)] * 999.0 for k in range(4)]

        def j_body(j, _):
            for k in range(2):
                obuf[j, pl.ds(16 * k, 16)] = qa[j, pl.ds(64 + 16 * k, 16)]
                obuf[j, pl.ds(32 + 16 * k, 16)] = qb[j, pl.ds(64 + 16 * k, 16)]
            for k in range(4):
                sl = pl.ds(16 * k, 16)
                v = (jnp.maximum(g1[j, sl] + qb[j, sl] + c999[k], 0.0)
                     + jnp.maximum(g3[j, sl] + qa[j, sl] + c999[k], 0.0))
                obuf[j, pl.ds(64 + 16 * k, 16)] = v
            return 0

        lax.fori_loop(0, rows, j_body, 0)
        pltpu.sync_copy(obuf, out_hbm.at[pl.ds(base, rows)])

    return assemble_kernel


def _node_body(agg, cn, wn2, w1, w2, be, p1o, p2o, nho):
    nh = jnp.maximum(
        jnp.dot(agg[...], wn2[...], preferred_element_type=jnp.float32)
        + cn[...], 0.0)
    nho[...] = nh
    p1o[...] = jnp.dot(nh, w1[...], preferred_element_type=jnp.float32)
    p2o[...] = jnp.dot(nh, w2[...], preferred_element_type=jnp.float32) + be[...]


def _cn_body(nf, wn1, bn, cno):
    cno[...] = jnp.dot(nf[...], wn1[...],
                       preferred_element_type=jnp.float32) + bn[...]


def _mlp_body(x, W0, b0, W1, b1, W2, b2, W3, b3, W4, b4, W5, b5, out):
    h = x[...]
    for W, b in ((W0, b0), (W1, b1), (W2, b2), (W3, b3), (W4, b4)):
        h = jax.nn.relu(jnp.dot(h, W[...], preferred_element_type=jnp.float32)
                        + b[...])
    logits = jnp.dot(h, W5[...], preferred_element_type=jnp.float32) + b5[...]
    best = logits[:, 0:1]
    idx = jnp.zeros_like(best, dtype=jnp.int32)
    for k in range(1, 4):
        cur = logits[:, k:k + 1]
        m = cur > best
        idx = jnp.where(m, k, idx)
        best = jnp.where(m, cur, best)
    out[...] = idx


def kernel(mol_a_node_features, mol_a_edge_features, mol_a_edges, mol_a_batch_indices,
           mol_b_node_features, mol_b_edge_features, mol_b_edges, mol_b_batch_indices,
           proposed_bonds, W_edge, b_edge, W_node, b_node,
           mlp_W0, mlp_b0, mlp_W1, mlp_b1, mlp_W2, mlp_b2,
           mlp_W3, mlp_b3, mlp_W4, mlp_b4, mlp_W5, mlp_b5):
    Na = mol_a_node_features.shape[0]
    Ea = mol_a_edges.shape[0]
    Eb = mol_b_edges.shape[0]
    nc = proposed_bonds.shape[1]
    E = Ea + Eb + 2 * nc

    pb0 = proposed_bonds[0].astype(jnp.int32)
    pb1 = (proposed_bonds[1] + Na).astype(jnp.int32)
    src = jnp.concatenate([mol_a_edges[:, 0].astype(jnp.int32),
                           mol_b_edges[:, 0].astype(jnp.int32) + Na, pb0, pb1])
    dst = jnp.concatenate([mol_a_edges[:, 1].astype(jnp.int32),
                           mol_b_edges[:, 1].astype(jnp.int32) + Na, pb1, pb0])
    ef = jnp.concatenate([mol_a_edge_features[:, 0], mol_b_edge_features[:, 0],
                          jnp.full((2 * nc,), 999.0, dtype=jnp.float32)])
    nf = jnp.concatenate([mol_a_node_features, mol_b_node_features], axis=0)
    N0 = nf.shape[0]
    N = ((N0 + NW * 16 - 1) // (NW * 16)) * (NW * 16)
    nf = jnp.concatenate(
        [nf, jnp.zeros((N - N0, nf.shape[1]), nf.dtype)], axis=0)
    rows_per_tile = N // NW

    perm = jnp.argsort(dst)
    dst_s = dst[perm]
    src_s = src[perm]
    ef_s = ef[perm]
    ep = ((E + EBLK - 1) // EBLK) * EBLK + EBLK
    pad = ep - E
    src_s = jnp.concatenate([src_s, jnp.zeros((pad,), jnp.int32)])
    dst_s = jnp.concatenate([dst_s, jnp.full((pad,), N - 1, jnp.int32)])
    ef_s = jnp.concatenate([ef_s, jnp.zeros((pad,), jnp.float32)])
    bounds = jnp.searchsorted(
        dst_s[:E], (rows_per_tile // 2) * jnp.arange(2 * NW + 1, dtype=jnp.int32)
    ).astype(jnp.int32)
    bounds = jnp.concatenate(
        [bounds, jnp.full((80 - (2 * NW + 1),), E, jnp.int32)])

    w64 = W_edge[64]
    W1 = jnp.concatenate(
        [W_edge[:32], jnp.zeros((32, 64), jnp.float32)], axis=1)
    W2 = W_edge[32:64]
    Wn1 = W_node[:5]
    Wn2 = W_node[5:]
    be2d = b_edge[None, :]
    bn2d = b_node[None, :]
    zero_tile = jnp.zeros((rows_per_tile // 2, 64), jnp.float32)

    RB = rows_per_tile * 4
    ng = N // RB
    cn = pl.pallas_call(
        _cn_body,
        grid=(ng,),
        in_specs=[pl.BlockSpec((RB, 5), lambda i: (i, 0)),
                  pl.BlockSpec((5, 32), lambda i: (0, 0)),
                  pl.BlockSpec((1, 32), lambda i: (0, 0))],
        out_specs=pl.BlockSpec((RB, 32), lambda i: (i, 0)),
        out_shape=jax.ShapeDtypeStruct((N, 32), jnp.float32),
    )(nf, Wn1, bn2d)

    node_call = pl.pallas_call(
        _node_body,
        grid=(ng,),
        in_specs=[pl.BlockSpec((RB, 64), lambda i: (i, 0)),
                  pl.BlockSpec((RB, 32), lambda i: (i, 0)),
                  pl.BlockSpec((64, 32), lambda i: (0, 0)),
                  pl.BlockSpec((32, 128), lambda i: (0, 0)),
                  pl.BlockSpec((32, 64), lambda i: (0, 0)),
                  pl.BlockSpec((1, 64), lambda i: (0, 0))],
        out_specs=[pl.BlockSpec((RB, 128), lambda i: (i, 0)),
                   pl.BlockSpec((RB, 64), lambda i: (i, 0)),
                   pl.BlockSpec((RB, 32), lambda i: (i, 0))],
        out_shape=[jax.ShapeDtypeStruct((N, 128), jnp.float32),
                   jax.ShapeDtypeStruct((N, 64), jnp.float32),
                   jax.ShapeDtypeStruct((N, 32), jnp.float32)],
    )

    edge_call = _make_edge_kernel(N, ep, rows_per_tile)
    assemble_call = _make_assemble_kernel(nc)

    P1_0 = jnp.zeros((N, 128), jnp.float32)
    P2b_0 = jnp.broadcast_to(b_edge, (N, 64)).astype(jnp.float32)

    def step(carry, _):
        P1, P2b = carry
        agg = edge_call(P1, P2b, src_s, dst_s, ef_s, bounds, zero_tile, w64)
        P1n, P2bn, _nh = node_call(agg, cn, Wn2, W1, W2, be2d)
        return (P1n, P2bn), None

    (P1f, P2bf), _ = lax.scan(step, (P1_0, P2b_0), None, length=NUM_STEPS - 1)

    agg = edge_call(P1f, P2bf, src_s, dst_s, ef_s, bounds, zero_tile, w64)
    _, _, nh_final = node_call(agg, cn, Wn2, W1, W2, be2d)

    def _pack_body(p2b, nh, out):
        out[...] = jnp.concatenate(
            [p2b[...], nh[...],
             jnp.zeros((p2b.shape[0], 32), jnp.float32)], axis=1)

    q = pl.pallas_call(
        _pack_body,
        grid=(ng,),
        in_specs=[pl.BlockSpec((RB, 64), lambda i: (i, 0)),
                  pl.BlockSpec((RB, 32), lambda i: (i, 0))],
        out_specs=pl.BlockSpec((RB, 128), lambda i: (i, 0)),
        out_shape=jax.ShapeDtypeStruct((N, 128), jnp.float32),
    )(P2bf, nh_final)

    mlp_in = assemble_call(P1f, q, pb0, pb1, w64)

    labels = pl.pallas_call(
        _mlp_body,
        out_shape=jax.ShapeDtypeStruct((nc, 1), jnp.int32),
    )(mlp_in, mlp_W0, mlp_b0, mlp_W1, mlp_b1, mlp_W2, mlp_b2,
      mlp_W3, mlp_b3, mlp_W4, mlp_b4, mlp_W5, mlp_b5)

    return jnp.concatenate([proposed_bonds.T.astype(labels.dtype), labels],
                           axis=1)
